# baseline (speedup 1.0000x reference)
import functools

import jax
import jax.numpy as jnp
from jax import lax
from jax.experimental import pallas as pl
from jax.experimental.pallas import tpu as pltpu
from jax.experimental.pallas import tpu_sc as plsc

D = 64
BATCH = 4096
HIST = 200
B = BATCH * HIST
NC, NS = 2, 16
NW = NC * NS
L = 16
XROWS_W = BATCH // NW
B_PER_W = XROWS_W * HIST
GR = 4
GROUP = GR * HIST
NG = XROWS_W // GR

_mesh = plsc.VectorSubcoreMesh(core_axis_name="c", subcore_axis_name="s")


@functools.partial(
    pl.kernel,
    mesh=_mesh,
    out_type=jax.ShapeDtypeStruct((B, 2 * D), jnp.float32),
    compiler_params=pltpu.CompilerParams(use_tc_tiling_on_sc=False),
    scratch_types=[
        pltpu.VMEM((XROWS_W, HIST), jnp.int32),
        pltpu.VMEM((GROUP, D), jnp.float32),
        pltpu.VMEM((GROUP, D), jnp.float32),
        pltpu.SemaphoreType.DMA,
        pltpu.SemaphoreType.DMA,
        pltpu.SemaphoreType.DMA,
        pltpu.SemaphoreType.DMA,
    ],
)
def _sc_gather(x_hbm, table_hbm, out_hbm, idx_v, rows0, rows1,
               gs0, gs1, ws0, ws1):
    wid = lax.axis_index("s") * NC + lax.axis_index("c")
    base = wid * B_PER_W
    rows = (rows0, rows1)
    gs = (gs0, gs1)
    ws = (ws0, ws1)

    pltpu.sync_copy(x_hbm.at[pl.ds(wid * XROWS_W, XROWS_W), :], idx_v)

    def for_group(g, b, fn):
        for q in range(GR):
            fn(pltpu.make_async_copy(
                table_hbm.at[idx_v.at[g * GR + q]],
                rows[b].at[pl.ds(q * HIST, HIST), :], gs[b]))

    def start_group(g, b):
        for_group(g, b, lambda cp: cp.start())

    def wait_group(g, b):
        for_group(g, b, lambda cp: cp.wait())

    start_group(0, 0)
    start_group(1, 1)

    def outer(j, carry):
        for b in range(2):
            g = 2 * j + b
            out_slc = out_hbm.at[pl.ds(base + g * GROUP, GROUP), pl.ds(0, D)]
            wait_group(g, b)
            pltpu.async_copy(rows[b], out_slc, ws[b])

            @pl.when(j < NG // 2 - 1)
            def _():
                pltpu.make_async_copy(rows[b], out_slc, ws[b]).wait()
                start_group(g + 2, b)

        return carry

    lax.fori_loop(0, NG // 2, outer, 0)

    for b in range(2):
        g = NG - 2 + b
        pltpu.make_async_copy(
            rows[b],
            out_hbm.at[pl.ds(base + g * GROUP, GROUP), pl.ds(0, D)],
            ws[b]).wait()


VOCAB = 1000000
TC = 128
TCH = TC * D // (2 * D)
NFULL = VOCAB // TC
NUNIF = NFULL // NW
TAILC = VOCAB % TC


@functools.partial(
    pl.kernel,
    mesh=_mesh,
    out_type=jax.ShapeDtypeStruct((VOCAB * D // (2 * D * 8), 8, 2 * D),
                                  jnp.float32),
    compiler_params=pltpu.CompilerParams(
        use_tc_tiling_on_sc=True, needs_layout_passes=False),
    scratch_types=[
        pltpu.VMEM((8, 8, 2 * D), jnp.float32),
        pltpu.VMEM((8, 8, 2 * D), jnp.float32),
        pltpu.VMEM((8, 8, 2 * D), jnp.float32),
        pltpu.VMEM((8, 8, 2 * D), jnp.float32),
        pltpu.SemaphoreType.DMA,
        pltpu.SemaphoreType.DMA,
        pltpu.SemaphoreType.DMA,
        pltpu.SemaphoreType.DMA,
    ],
)
def _sc_transpose(wt_hbm, tail_hbm, out_hbm, src0, src1, buf0, buf1,
                  is0, is1, os0, os1):
    wid = lax.axis_index("s") * NC + lax.axis_index("c")
    src = (src0, src1)
    buf = (buf0, buf1)
    isem = (is0, is1)
    osem = (os0, os1)
    lane = lax.iota(jnp.int32, L)

    def in_cp(k, p):
        c = wid + NW * k
        return pltpu.make_async_copy(
            wt_hbm.at[:, :, pl.ds(c * TC, TC)], src[p], isem[p])

    def out_cp(k, p):
        c = wid + NW * k
        return pltpu.make_async_copy(
            buf[p], out_hbm.at[pl.ds(c * 8, 8), :, :], osem[p])

    def transpose_chunk(p):
        def row(fr, carry):
            for g in range(8):
                d_vec = lane + (L * g) % D
                r_vec = jnp.full((L,), 2 * fr + g // 4, jnp.int32)
                buf[p][fr // 8, fr % 8, pl.ds(L * g, L)] = plsc.load_gather(
                    src[p], [d_vec // 8, d_vec % 8, r_vec])
            return carry
        lax.fori_loop(0, TCH, row, 0)

    in_cp(0, 0).start()
    in_cp(1, 1).start()
    for k0 in (0, 1):
        in_cp(k0, k0).wait()
        transpose_chunk(k0)
        out_cp(k0, k0).start()
        in_cp(k0 + 2, k0).start()

    def outer(j, carry):
        for p in (0, 1):
            k = 2 * j + 2 + p
            in_cp(k, p).wait()
            out_cp(k - 2, p).wait()
            transpose_chunk(p)
            out_cp(k, p).start()

            @pl.when(k + 2 < NUNIF)
            def _():
                in_cp(k + 2, p).start()

        return carry

    lax.fori_loop(0, (NUNIF - 2) // 2, outer, 0)

    out_cp(NUNIF - 2, 0).wait()
    out_cp(NUNIF - 1, 1).wait()

    @pl.when(wid < NFULL - NW * NUNIF)
    def _():
        c = NW * NUNIF + wid
        pltpu.sync_copy(wt_hbm.at[:, :, pl.ds(c * TC, TC)], src0)
        transpose_chunk(0)
        pltpu.sync_copy(buf0, out_hbm.at[pl.ds(c * 8, 8), :, :])

    @pl.when(wid == NFULL - NW * NUNIF)
    def _():
        pltpu.sync_copy(tail_hbm, src0.at[pl.ds(0, 4), :, :])
        pltpu.sync_copy(src0.at[pl.ds(0, 4), :, :],
                        out_hbm.at[pl.ds(NFULL * 8, 4), :, :])


def kernel(x, weight):
    wt3 = weight.T.reshape(8, 8, VOCAB)
    tail3 = weight[VOCAB - TAILC:].reshape(4, 8, 2 * D)
    table = _sc_transpose(wt3, tail3).reshape(VOCAB, D)
    out = _sc_gather(x, table)
    return out[:, :D].reshape(BATCH, HIST, D)


# speedup vs baseline: 1.9689x; 1.9689x over previous
import functools

import jax
import jax.numpy as jnp
from jax import lax
from jax.experimental import pallas as pl
from jax.experimental.pallas import tpu as pltpu
from jax.experimental.pallas import tpu_sc as plsc

D = 64
BATCH = 4096
HIST = 200
B = BATCH * HIST
NC, NS = 2, 16
NW = NC * NS
XROWS_W = BATCH // NW
B_PER_W = XROWS_W * HIST
GR = 4
GROUP = GR * HIST
NG = XROWS_W // GR

_mesh = plsc.VectorSubcoreMesh(core_axis_name="c", subcore_axis_name="s")


@functools.partial(
    pl.kernel,
    mesh=_mesh,
    out_type=jax.ShapeDtypeStruct((B, 2 * D), jnp.float32),
    compiler_params=pltpu.CompilerParams(use_tc_tiling_on_sc=False),
    scratch_types=[
        pltpu.VMEM((XROWS_W, HIST), jnp.int32),
        pltpu.VMEM((GROUP, D), jnp.float32),
        pltpu.VMEM((GROUP, D), jnp.float32),
        pltpu.SemaphoreType.DMA,
        pltpu.SemaphoreType.DMA,
        pltpu.SemaphoreType.DMA,
        pltpu.SemaphoreType.DMA,
    ],
)
def _sc_gather(x_hbm, table_hbm, out_hbm, idx_v, rows0, rows1,
               gs0, gs1, ws0, ws1):
    wid = lax.axis_index("s") * NC + lax.axis_index("c")
    base = wid * B_PER_W
    rows = (rows0, rows1)
    gs = (gs0, gs1)
    ws = (ws0, ws1)

    pltpu.sync_copy(x_hbm.at[pl.ds(wid * XROWS_W, XROWS_W), :], idx_v)

    def for_group(g, b, fn):
        for q in range(GR):
            fn(pltpu.make_async_copy(
                table_hbm.at[idx_v.at[g * GR + q]],
                rows[b].at[pl.ds(q * HIST, HIST), :], gs[b]))

    def start_group(g, b):
        for_group(g, b, lambda cp: cp.start())

    def wait_group(g, b):
        for_group(g, b, lambda cp: cp.wait())

    start_group(0, 0)
    start_group(1, 1)

    def outer(j, carry):
        for b in range(2):
            g = 2 * j + b
            out_slc = out_hbm.at[pl.ds(base + g * GROUP, GROUP), pl.ds(0, D)]
            wait_group(g, b)
            pltpu.async_copy(rows[b], out_slc, ws[b])

            @pl.when(j < NG // 2 - 1)
            def _():
                pltpu.make_async_copy(rows[b], out_slc, ws[b]).wait()
                start_group(g + 2, b)

        return carry

    lax.fori_loop(0, NG // 2, outer, 0)

    for b in range(2):
        g = NG - 2 + b
        pltpu.make_async_copy(
            rows[b],
            out_hbm.at[pl.ds(base + g * GROUP, GROUP), pl.ds(0, D)],
            ws[b]).wait()


def kernel(x, weight):
    out = _sc_gather(x, weight)
    return out[:, :D].reshape(BATCH, HIST, D)
